# baseline (speedup 1.0000x reference)
import jax
import jax.numpy as jnp
from jax import lax
from jax.experimental import pallas as pl
from jax.experimental.pallas import tpu as pltpu

_TN = 512
_SG = 256
_GPAD = 10240
_GR = 1024
_NB = 8
_LOOPS = 3
_EPSV = 1e-07
_NEG = -1e30


def _lstm_body(ht_ref, ct_ref, rt_ref, s8_ref, wa_ref, wb_ref, bs_ref,
               hn_ref, cn_ref):
    nh = ht_ref.shape[0]
    ht = ht_ref[...]
    ct = ct_ref[...]
    s = s8_ref[:1, :]
    rt = rt_ref[...] / (s + _EPSV)
    gates = (lax.dot_general(wa_ref[...], ht, (((0,), (0,)), ((), ())),
                             preferred_element_type=jnp.float32)
             + lax.dot_general(wb_ref[...], rt, (((0,), (0,)), ((), ())),
                               preferred_element_type=jnp.float32)
             + bs_ref[...])
    i_g = jax.nn.sigmoid(gates[0 * nh:1 * nh, :])
    f_g = jax.nn.sigmoid(gates[1 * nh:2 * nh, :])
    g_g = jnp.tanh(gates[2 * nh:3 * nh, :])
    o_g = jax.nn.sigmoid(gates[3 * nh:4 * nh, :])
    c_new = f_g * ct + i_g * g_g
    hn_ref[...] = o_g * jnp.tanh(c_new)
    cn_ref[...] = c_new


def _seg_body(t0_ref, nt_ref, ht_ref, wmt_ref, bmr_ref, bmc_ref,
              feat_hbm, seg_hbm, rt_ref, s8_ref,
              ftacc_ref, fbuf_ref, gbuf_ref, sem_ref):
    s = pl.program_id(0)
    t0 = t0_ref[s]
    nt = nt_ref[s]
    g0 = s * _SG

    wh = jnp.dot(wmt_ref[...], ht_ref[...], preferred_element_type=jnp.float32)
    eb = jnp.dot(bmr_ref[...], ht_ref[...], preferred_element_type=jnp.float32)

    ftacc_ref[...] = jnp.zeros(ftacc_ref.shape, jnp.float32)

    def _issue(k):
        t = t0 + k
        buf = lax.rem(k, _NB)
        pltpu.make_async_copy(feat_hbm.at[pl.ds(t * _TN, _TN), :],
                              fbuf_ref.at[buf], sem_ref.at[buf, 0]).start()
        pltpu.make_async_copy(seg_hbm.at[pl.ds(t * _TN, _TN), :],
                              gbuf_ref.at[buf], sem_ref.at[buf, 1]).start()

    def _prime(k, _):
        _issue(k)
        return 0

    lax.fori_loop(0, jnp.minimum(nt, _NB - 1), _prime, 0)

    lane = lax.broadcasted_iota(jnp.int32, (_TN, _SG), 1)

    def _chunk(k, carry):
        m_run, s_run = carry
        buf = lax.rem(k, _NB)

        @pl.when(k + _NB - 1 < nt)
        def _prefetch():
            _issue(k + _NB - 1)

        t = t0 + k
        pltpu.make_async_copy(feat_hbm.at[pl.ds(t * _TN, _TN), :],
                              fbuf_ref.at[buf], sem_ref.at[buf, 0]).wait()
        pltpu.make_async_copy(seg_hbm.at[pl.ds(t * _TN, _TN), :],
                              gbuf_ref.at[buf], sem_ref.at[buf, 1]).wait()

        f = fbuf_ref[buf]
        rel = gbuf_ref[buf] - g0
        oneb = rel == lane

        e_full = jnp.dot(f, wh, preferred_element_type=jnp.float32) + eb
        e_m = jnp.where(oneb, e_full, _NEG)
        cmax = jnp.max(e_m, axis=0, keepdims=True)
        new_m = jnp.maximum(m_run, cmax)
        p_mat = jnp.where(oneb, jnp.exp(e_full - new_m), 0.0)
        scale = jnp.exp(m_run - new_m)
        s_new = s_run * scale + jnp.sum(p_mat, axis=0, keepdims=True)
        ftacc_ref[...] = (ftacc_ref[...] * scale
                          + lax.dot_general(f, p_mat, (((0,), (0,)), ((), ())),
                                            preferred_element_type=jnp.float32))
        return new_m, s_new

    m_fin, s_fin = lax.fori_loop(
        0, nt, _chunk,
        (jnp.full((1, _SG), _NEG, jnp.float32),
         jnp.zeros((1, _SG), jnp.float32)))

    rt_ref[...] = (lax.dot_general(wmt_ref[...], ftacc_ref[...],
                                   (((0,), (0,)), ((), ())),
                                   preferred_element_type=jnp.float32)
                   + bmc_ref[...] * s_fin)
    s8_ref[...] = jnp.broadcast_to(s_fin, s8_ref.shape)


def kernel(features, feature_graph_index, W_m, b_m, W_ih, W_hh, b_ih, b_hh):
    n, d = features.shape
    h_dim = W_hh.shape[1]
    seg = feature_graph_index.astype(jnp.int32)
    nstep = _GPAD // _SG

    bounds = jnp.arange(0, _GPAD + 1, _SG, dtype=jnp.int32)
    off = jnp.sum((seg[:, None] < bounds[None, :]).astype(jnp.int32),
                  axis=0).astype(jnp.int32)
    off_lo = off[:-1]
    off_hi = off[1:]
    t0s = off_lo // _TN
    t1s = (off_hi + _TN - 1) // _TN
    nts = jnp.where(off_hi > off_lo, t1s - t0s, 0).astype(jnp.int32)
    t0s = t0s.astype(jnp.int32)

    wmt = W_m.T
    bmr = b_m.reshape(1, h_dim)
    bmc = b_m.reshape(h_dim, 1)
    w_iht = W_ih.T
    wa = w_iht[:h_dim, :] + W_hh.T
    wb = w_iht[h_dim:, :]
    bs = (b_ih + b_hh).reshape(4 * h_dim, 1)
    seg2 = seg.reshape(n, 1)

    lstm_call = pl.pallas_call(
        _lstm_body,
        grid=(_GPAD // _GR,),
        in_specs=[
            pl.BlockSpec((h_dim, _GR), lambda i: (0, i)),
            pl.BlockSpec((h_dim, _GR), lambda i: (0, i)),
            pl.BlockSpec((h_dim, _GR), lambda i: (0, i)),
            pl.BlockSpec((8, _GR), lambda i: (0, i)),
            pl.BlockSpec((h_dim, 4 * h_dim), lambda i: (0, 0)),
            pl.BlockSpec((h_dim, 4 * h_dim), lambda i: (0, 0)),
            pl.BlockSpec((4 * h_dim, 1), lambda i: (0, 0)),
        ],
        out_specs=[
            pl.BlockSpec((h_dim, _GR), lambda i: (0, i)),
            pl.BlockSpec((h_dim, _GR), lambda i: (0, i)),
        ],
        out_shape=[
            jax.ShapeDtypeStruct((h_dim, _GPAD), jnp.float32),
            jax.ShapeDtypeStruct((h_dim, _GPAD), jnp.float32),
        ],
    )

    seg_call = pl.pallas_call(
        _seg_body,
        grid_spec=pltpu.PrefetchScalarGridSpec(
            num_scalar_prefetch=2,
            grid=(nstep,),
            in_specs=[
                pl.BlockSpec((h_dim, _SG), lambda s, t0, nt: (0, s)),
                pl.BlockSpec((d, h_dim), lambda s, t0, nt: (0, 0)),
                pl.BlockSpec((1, h_dim), lambda s, t0, nt: (0, 0)),
                pl.BlockSpec((h_dim, 1), lambda s, t0, nt: (0, 0)),
                pl.BlockSpec(memory_space=pltpu.MemorySpace.HBM),
                pl.BlockSpec(memory_space=pltpu.MemorySpace.HBM),
            ],
            out_specs=[
                pl.BlockSpec((h_dim, _SG), lambda s, t0, nt: (0, s)),
                pl.BlockSpec((8, _SG), lambda s, t0, nt: (0, s)),
            ],
            scratch_shapes=[
                pltpu.VMEM((d, _SG), jnp.float32),
                pltpu.VMEM((_NB, _TN, d), jnp.float32),
                pltpu.VMEM((_NB, _TN, 1), jnp.int32),
                pltpu.SemaphoreType.DMA((_NB, 2)),
            ],
        ),
        out_shape=[
            jax.ShapeDtypeStruct((h_dim, _GPAD), jnp.float32),
            jax.ShapeDtypeStruct((8, _GPAD), jnp.float32),
        ],
    )

    ht = jnp.zeros((h_dim, _GPAD), jnp.float32)
    ct = jnp.zeros((h_dim, _GPAD), jnp.float32)
    rt = jnp.zeros((h_dim, _GPAD), jnp.float32)
    s8 = jnp.zeros((8, _GPAD), jnp.float32)

    for _ in range(_LOOPS):
        ht, ct = lstm_call(ht, ct, rt, s8, wa, wb, bs)
        rt, s8 = seg_call(t0s, nts, ht, wmt, bmr, bmc, features, seg2)

    g_num = 10000
    h_fin = ht[:, :g_num].T
    r_fin = (rt[:, :g_num] / (s8[:1, :g_num] + _EPSV)).T
    return jnp.concatenate([h_fin, r_fin], axis=-1)
